# baseline (speedup 1.0000x reference)
import functools

import jax
import jax.numpy as jnp
import numpy as np
from jax import lax
from jax.experimental import pallas as pl
from jax.experimental.pallas import tpu as pltpu
from jax.experimental.pallas import tpu_sc as plsc

N = 10000
E = 320000
H = 128
ED = 16
T = 10
G = 64

NC = 2
NS = 16
NW = NC * NS
LN = 16
CH = H // LN
CB = H // 32

EB = 96
NRING = 3
UE = 8
BLOCKS_PER_TILE = NRING * (-(-E // (NW * EB * NRING)))
E_PAD = NW * EB * BLOCKS_PER_TILE

NP = NS * 8 * (-(-(N + 1) // (NS * 8)))
ZR = NP // NS
FR = NP // NS

RB = 2000


def _lane_bcast(av, k):
    return lax.gather(
        av, jnp.full((LN, 1), k, jnp.int32),
        lax.GatherDimensionNumbers(
            offset_dims=(), collapsed_slice_dims=(0,), start_index_map=(0,)),
        slice_sizes=(1,),
        mode=lax.GatherScatterMode.PROMISE_IN_BOUNDS)


def _compute_block(rows_v, attr_v, ew_v, ebc):
    def edge_body(j, carry2):
        e0 = j * UE
        ar = j // (8 // UE)
        ac = (j % (8 // UE)) * (UE * ED)
        avs = [attr_v[ar, pl.ds(ac + u * ED, ED)] for u in range(UE)]
        accs = [[ebc[cb] for cb in range(CB)] for u in range(UE)]
        for k in range(ED):
            wk = [plsc.bitcast(ew_v[k, pl.ds(LN * cb, LN)], jnp.bfloat16)
                  for cb in range(CB)]
            for u in range(UE):
                skf = _lane_bcast(avs[u], k)
                skb = plsc.pack(skf, skf,
                                format=plsc.PackFormat.INTERLEAVED)
                for cb in range(CB):
                    accs[u][cb] = accs[u][cb] + skb * wk[cb]
        for u in range(UE):
            for cb in range(CB):
                lo, hi = plsc.unpack(
                    accs[u][cb], format=plsc.PackFormat.INTERLEAVED)
                base = 32 * cb
                m0 = jnp.maximum(rows_v[e0 + u, pl.ds(base, LN)] + lo, 0.0)
                m1 = jnp.maximum(
                    rows_v[e0 + u, pl.ds(base + LN, LN)] + hi, 0.0)
                rows_v[e0 + u, pl.ds(base, LN)] = m0
                rows_v[e0 + u, pl.ds(base + LN, LN)] = m1
        return carry2

    lax.fori_loop(0, EB // UE, edge_body, 0)


def _edge_body(xn_hbm, idx_hbm, attr_hbm, ew_hbm, ebias_hbm, zero_hbm,
               out_hbm,
               idx_v0, idx_v1, idx_v2,
               attr_v0, attr_v1, attr_v2,
               rows_v0, rows_v1, rows_v2, ew_v, ebias_v,
               gsem0, gsem1, gsem2, asem0, asem1, asem2,
               ssem0, ssem1, ssem2, isem0, isem1, isem2, aggr_sh):
    idx_v = (idx_v0, idx_v1, idx_v2)
    attr_v = (attr_v0, attr_v1, attr_v2)
    rows_v = (rows_v0, rows_v1, rows_v2)
    gsem = (gsem0, gsem1, gsem2)
    asem = (asem0, asem1, asem2)
    ssem = (ssem0, ssem1, ssem2)
    isem = (isem0, isem1, isem2)
    c = lax.axis_index("c")
    s = lax.axis_index("s")
    wid = s * NC + c

    pltpu.sync_copy(ew_hbm, ew_v)
    pltpu.sync_copy(ebias_hbm, ebias_v)
    pltpu.sync_copy(zero_hbm, aggr_sh.at[pl.ds(s * ZR, ZR)])
    plsc.subcore_barrier()

    ebc = [plsc.bitcast(ebias_v[pl.ds(LN * cb, LN)], jnp.bfloat16)
           for cb in range(CB)]

    def start_idx(b, r):
        return pltpu.async_copy(idx_hbm.at[wid, b], idx_v[r], isem[r])

    def start_attr(b, r):
        return pltpu.async_copy(attr_hbm.at[wid, b], attr_v[r], asem[r])

    def start_gather(r):
        return pltpu.async_copy(xn_hbm.at[idx_v[r].at[0]], rows_v[r], gsem[r])

    def start_scatter(r):
        return pltpu.async_copy(
            rows_v[r], aggr_sh.at[idx_v[r].at[1]], ssem[r], add=True)

    def wait(sem, ref_pair):
        pltpu.make_async_copy(ref_pair[0], ref_pair[1], sem).wait()

    start_idx(0, 0)
    start_idx(1, 1)
    start_attr(0, 0)
    start_attr(1, 1)
    wait(isem[0], (idx_hbm.at[wid, 0], idx_v[0]))
    start_gather(0)

    def group_body(g, carry):
        for u in range(NRING):
            b = g * NRING + u
            nxt = (u + 1) % NRING
            nxt2 = (u + 2) % NRING
            wait(asem[u], (attr_hbm.at[wid, b], attr_v[u]))
            wait(gsem[u], (xn_hbm.at[idx_v[u].at[0]], rows_v[u]))

            @pl.when(b >= 2)
            def _():
                wait(ssem[nxt], (rows_v[nxt], aggr_sh.at[idx_v[nxt].at[1]]))

            @pl.when(b + 1 < BLOCKS_PER_TILE)
            def _():
                wait(isem[nxt], (idx_hbm.at[wid, b], idx_v[nxt]))
                start_gather(nxt)

            @pl.when(b + 2 < BLOCKS_PER_TILE)
            def _():
                start_idx(b + 2, nxt2)
                start_attr(b + 2, nxt2)

            _compute_block(rows_v[u], attr_v[u], ew_v, ebc)
            start_scatter(u)
        return carry

    lax.fori_loop(0, BLOCKS_PER_TILE // NRING, group_body, 0)
    for b in (BLOCKS_PER_TILE - 2, BLOCKS_PER_TILE - 1):
        r = b % NRING
        wait(ssem[r], (rows_v[r], aggr_sh.at[idx_v[r].at[1]]))
    plsc.subcore_barrier()
    pltpu.sync_copy(aggr_sh.at[pl.ds(s * FR, FR)],
                    out_hbm.at[c, pl.ds(s * FR, FR)])


@functools.cache
def _edge_call():
    return pl.kernel(
        _edge_body,
        out_type=jax.ShapeDtypeStruct((NC, NP, H), jnp.float32),
        mesh=plsc.VectorSubcoreMesh(core_axis_name="c", subcore_axis_name="s",
                                    num_cores=NC, num_subcores=NS),
        compiler_params=pltpu.CompilerParams(needs_layout_passes=False),
        scratch_types=(
            [pltpu.VMEM((2, EB), jnp.int32) for _ in range(NRING)]
            + [pltpu.VMEM((EB * ED // 128, 128), jnp.float32)
               for _ in range(NRING)]
            + [pltpu.VMEM((EB, H), jnp.float32)
               for _ in range(NRING)]
            + [pltpu.VMEM((ED, H // 2), jnp.float32),
               pltpu.VMEM((H // 2,), jnp.float32)]
            + [pltpu.SemaphoreType.DMA for _ in range(4 * NRING)]
            + [pltpu.VMEM_SHARED((NP, H), jnp.float32)]
        ),
    )


def _node_lin_body(x_ref, w_ref, b_ref, o_ref):
    o_ref[...] = jnp.dot(x_ref[...], w_ref[...],
                         preferred_element_type=jnp.float32) + b_ref[...]


def _node_linear(x, w, b):
    fin = x.shape[1]
    return pl.pallas_call(
        _node_lin_body,
        grid=(N // RB,),
        in_specs=[pl.BlockSpec((RB, fin), lambda i: (i, 0)),
                  pl.BlockSpec((fin, H), lambda i: (0, 0)),
                  pl.BlockSpec((1, H), lambda i: (0, 0))],
        out_specs=pl.BlockSpec((RB, H), lambda i: (i, 0)),
        out_shape=jax.ShapeDtypeStruct((N, H), jnp.float32),
    )(x, w, b.reshape(1, H))


def _mlp_next_body(p_ref, xn_ref, m1_ref, b1_ref, m2_ref, b2_ref,
                   nw_ref, nb_ref, o_ref):
    aggr = p_ref[0] + p_ref[1] + jnp.maximum(xn_ref[...], 0.0)
    t = jnp.maximum(
        jnp.dot(aggr, m1_ref[...], preferred_element_type=jnp.float32)
        + b1_ref[...], 0.0)
    x1 = jnp.maximum(
        jnp.dot(t, m2_ref[...], preferred_element_type=jnp.float32)
        + b2_ref[...], 0.0)
    o_ref[...] = jnp.dot(x1, nw_ref[...],
                         preferred_element_type=jnp.float32) + nb_ref[...]


def _mlp_next(p, xn, m1, b1, m2, b2, nw, nb):
    return pl.pallas_call(
        _mlp_next_body,
        grid=(N // RB,),
        in_specs=[pl.BlockSpec((NC, RB, H), lambda i: (0, i, 0)),
                  pl.BlockSpec((RB, H), lambda i: (i, 0)),
                  pl.BlockSpec((H, H), lambda i: (0, 0)),
                  pl.BlockSpec((1, H), lambda i: (0, 0)),
                  pl.BlockSpec((H, H), lambda i: (0, 0)),
                  pl.BlockSpec((1, H), lambda i: (0, 0)),
                  pl.BlockSpec((H, H), lambda i: (0, 0)),
                  pl.BlockSpec((1, H), lambda i: (0, 0))],
        out_specs=pl.BlockSpec((RB, H), lambda i: (i, 0)),
        out_shape=jax.ShapeDtypeStruct((N, H), jnp.float32),
    )(p, xn, m1, b1.reshape(1, H), m2, b2.reshape(1, H), nw, nb.reshape(1, H))


def _mlp_pool_body(p_ref, xn_ref, m1_ref, b1_ref, m2_ref, b2_ref,
                   batch_ref, lw_ref, lb_ref, o_ref, acc_ref):
    i = pl.program_id(0)
    aggr = p_ref[0] + p_ref[1] + jnp.maximum(xn_ref[...], 0.0)
    t = jnp.maximum(
        jnp.dot(aggr, m1_ref[...], preferred_element_type=jnp.float32)
        + b1_ref[...], 0.0)
    x2 = jnp.maximum(
        jnp.dot(t, m2_ref[...], preferred_element_type=jnp.float32)
        + b2_ref[...], 0.0)
    bb = batch_ref[0, 0, :]
    onehot = (lax.broadcasted_iota(jnp.int32, (G, RB), 0)
              == bb[None, :]).astype(jnp.float32)
    pooled = jnp.dot(onehot, x2, preferred_element_type=jnp.float32)

    @pl.when(i == 0)
    def _():
        acc_ref[...] = jnp.zeros_like(acc_ref)

    acc_ref[...] += pooled

    @pl.when(i == N // RB - 1)
    def _():
        o_ref[...] = jnp.dot(acc_ref[...], lw_ref[...],
                             preferred_element_type=jnp.float32) + lb_ref[...]


def _mlp_pool(p, xn, m1, b1, m2, b2, batch3, lw, lb):
    return pl.pallas_call(
        _mlp_pool_body,
        grid=(N // RB,),
        in_specs=[pl.BlockSpec((NC, RB, H), lambda i: (0, i, 0)),
                  pl.BlockSpec((RB, H), lambda i: (i, 0)),
                  pl.BlockSpec((H, H), lambda i: (0, 0)),
                  pl.BlockSpec((1, H), lambda i: (0, 0)),
                  pl.BlockSpec((H, H), lambda i: (0, 0)),
                  pl.BlockSpec((1, H), lambda i: (0, 0)),
                  pl.BlockSpec((1, 1, RB), lambda i: (i, 0, 0)),
                  pl.BlockSpec((H, 1), lambda i: (0, 0)),
                  pl.BlockSpec((1, 1), lambda i: (0, 0))],
        out_specs=pl.BlockSpec((G, 1), lambda i: (0, 0)),
        out_shape=jax.ShapeDtypeStruct((G, 1), jnp.float32),
        scratch_shapes=[pltpu.VMEM((G, H), jnp.float32)],
    )(p, xn, m1, b1.reshape(1, H), m2, b2.reshape(1, H), batch3, lw,
      lb.reshape(1, 1))


def _shuffle_w(w):
    w4 = w.reshape(ED, CB, 2, LN)
    wb = w4.transpose(0, 1, 3, 2).reshape(ED, H // 2, 2).astype(jnp.bfloat16)
    return lax.bitcast_convert_type(wb, jnp.float32)


def _shuffle_b(b):
    b4 = b.reshape(CB, 2, LN)
    bb = b4.transpose(0, 2, 1).reshape(H // 2, 2).astype(jnp.bfloat16)
    return lax.bitcast_convert_type(bb, jnp.float32)


def kernel(pos, z, edge_index, edge_attr, batch,
           e1_node_W, e1_node_b, e1_edge_W, e1_edge_b,
           e1_m1_W, e1_m1_b, e1_m2_W, e1_m2_b,
           e2_node_W, e2_node_b, e2_edge_W, e2_edge_b,
           e2_m1_W, e2_m1_b, e2_m2_W, e2_m2_b,
           lin_W, lin_b):
    x0 = jnp.concatenate(
        [pos, jax.nn.one_hot(z, T, dtype=jnp.float32)], axis=1)
    src = edge_index[0]
    dst = edge_index[1]
    pad = E_PAD - E
    srcp = jnp.concatenate([src, jnp.zeros((pad,), src.dtype)]).reshape(
        NW, BLOCKS_PER_TILE, 1, EB)
    dstp = jnp.concatenate([dst, jnp.full((pad,), N, dst.dtype)]).reshape(
        NW, BLOCKS_PER_TILE, 1, EB)
    idxp = jnp.concatenate([srcp, dstp], axis=2)
    attrp = jnp.concatenate(
        [edge_attr, jnp.zeros((pad, ED), edge_attr.dtype)]).reshape(
        NW, BLOCKS_PER_TILE, EB * ED // 128, 128)
    zeros = jnp.zeros((ZR, H), jnp.float32)
    batch3 = batch.reshape(N // RB, 1, RB)

    xn1 = _node_linear(x0, e1_node_W, e1_node_b)
    p1 = _edge_call()(xn1, idxp, attrp, _shuffle_w(e1_edge_W),
                      _shuffle_b(e1_edge_b), zeros)
    xn2 = _mlp_next(p1, xn1, e1_m1_W, e1_m1_b, e1_m2_W, e1_m2_b,
                    e2_node_W, e2_node_b)
    p2 = _edge_call()(xn2, idxp, attrp, _shuffle_w(e2_edge_W),
                      _shuffle_b(e2_edge_b), zeros)
    return _mlp_pool(p2, xn2, e2_m1_W, e2_m1_b, e2_m2_W, e2_m2_b,
                     batch3, lin_W, lin_b)


# speedup vs baseline: 1.0229x; 1.0229x over previous
import functools

import jax
import jax.numpy as jnp
import numpy as np
from jax import lax
from jax.experimental import pallas as pl
from jax.experimental.pallas import tpu as pltpu
from jax.experimental.pallas import tpu_sc as plsc

N = 10000
E = 320000
H = 128
ED = 16
T = 10
G = 64

NC = 2
NS = 16
NW = NC * NS
LN = 16
CH = H // LN
CB = H // 32

EB = 96
NRING = 3
UE = 4
BLOCKS_PER_TILE = NRING * (-(-E // (NW * EB * NRING)))
E_PAD = NW * EB * BLOCKS_PER_TILE

NP = NS * 8 * (-(-(N + 1) // (NS * 8)))
ZR = NP // NS
FR = NP // NS

RB = 2000


def _lane_bcast(av, k):
    return lax.gather(
        av, jnp.full((LN, 1), k, jnp.int32),
        lax.GatherDimensionNumbers(
            offset_dims=(), collapsed_slice_dims=(0,), start_index_map=(0,)),
        slice_sizes=(1,),
        mode=lax.GatherScatterMode.PROMISE_IN_BOUNDS)


def _compute_block(rows_v, attr_v, ew_v, ebc):
    def edge_body(j, carry2):
        e0 = j * UE
        ar = j // (8 // UE)
        ac = (j % (8 // UE)) * (UE * ED)
        avs = [attr_v[ar, pl.ds(ac + u * ED, ED)] for u in range(UE)]
        accs = [[ebc[cb] for cb in range(CB)] for u in range(UE)]
        for k in range(ED):
            wk = [plsc.bitcast(ew_v[k, pl.ds(LN * cb, LN)], jnp.bfloat16)
                  for cb in range(CB)]
            for u in range(UE):
                skf = _lane_bcast(avs[u], k)
                skb = plsc.pack(skf, skf,
                                format=plsc.PackFormat.INTERLEAVED)
                for cb in range(CB):
                    accs[u][cb] = accs[u][cb] + skb * wk[cb]
        for u in range(UE):
            for cb in range(CB):
                lo, hi = plsc.unpack(
                    accs[u][cb], format=plsc.PackFormat.INTERLEAVED)
                base = 32 * cb
                m0 = jnp.maximum(rows_v[e0 + u, pl.ds(base, LN)] + lo, 0.0)
                m1 = jnp.maximum(
                    rows_v[e0 + u, pl.ds(base + LN, LN)] + hi, 0.0)
                rows_v[e0 + u, pl.ds(base, LN)] = m0
                rows_v[e0 + u, pl.ds(base + LN, LN)] = m1
        return carry2

    lax.fori_loop(0, EB // UE, edge_body, 0)


def _edge_body(xn_hbm, idx_hbm, attr_hbm, ew_hbm, ebias_hbm, zero_hbm,
               out_hbm,
               idx_v0, idx_v1, idx_v2,
               attr_v0, attr_v1, attr_v2,
               rows_v0, rows_v1, rows_v2, ew_v, ebias_v,
               gsem0, gsem1, gsem2, asem0, asem1, asem2,
               ssem0, ssem1, ssem2, isem0, isem1, isem2, aggr_sh):
    idx_v = (idx_v0, idx_v1, idx_v2)
    attr_v = (attr_v0, attr_v1, attr_v2)
    rows_v = (rows_v0, rows_v1, rows_v2)
    gsem = (gsem0, gsem1, gsem2)
    asem = (asem0, asem1, asem2)
    ssem = (ssem0, ssem1, ssem2)
    isem = (isem0, isem1, isem2)
    c = lax.axis_index("c")
    s = lax.axis_index("s")
    wid = s * NC + c

    pltpu.sync_copy(ew_hbm, ew_v)
    pltpu.sync_copy(ebias_hbm, ebias_v)
    pltpu.sync_copy(zero_hbm, aggr_sh.at[pl.ds(s * ZR, ZR)])
    plsc.subcore_barrier()

    ebc = [plsc.bitcast(ebias_v[pl.ds(LN * cb, LN)], jnp.bfloat16)
           for cb in range(CB)]

    def start_idx(b, r):
        return pltpu.async_copy(idx_hbm.at[wid, b], idx_v[r], isem[r])

    def start_attr(b, r):
        return pltpu.async_copy(attr_hbm.at[wid, b], attr_v[r], asem[r])

    def start_gather(r):
        return pltpu.async_copy(xn_hbm.at[idx_v[r].at[0]], rows_v[r], gsem[r])

    def start_scatter(r):
        return pltpu.async_copy(
            rows_v[r], aggr_sh.at[idx_v[r].at[1]], ssem[r], add=True)

    def wait(sem, ref_pair):
        pltpu.make_async_copy(ref_pair[0], ref_pair[1], sem).wait()

    start_idx(0, 0)
    start_idx(1, 1)
    start_attr(0, 0)
    start_attr(1, 1)
    wait(isem[0], (idx_hbm.at[wid, 0], idx_v[0]))
    start_gather(0)

    def group_body(g, carry):
        for u in range(NRING):
            b = g * NRING + u
            nxt = (u + 1) % NRING
            nxt2 = (u + 2) % NRING
            wait(asem[u], (attr_hbm.at[wid, b], attr_v[u]))
            wait(gsem[u], (xn_hbm.at[idx_v[u].at[0]], rows_v[u]))

            @pl.when(b >= 2)
            def _():
                wait(ssem[nxt], (rows_v[nxt], aggr_sh.at[idx_v[nxt].at[1]]))

            @pl.when(b + 1 < BLOCKS_PER_TILE)
            def _():
                wait(isem[nxt], (idx_hbm.at[wid, b], idx_v[nxt]))
                start_gather(nxt)

            @pl.when(b + 2 < BLOCKS_PER_TILE)
            def _():
                start_idx(b + 2, nxt2)
                start_attr(b + 2, nxt2)

            _compute_block(rows_v[u], attr_v[u], ew_v, ebc)
            start_scatter(u)
        return carry

    lax.fori_loop(0, BLOCKS_PER_TILE // NRING, group_body, 0)
    for b in (BLOCKS_PER_TILE - 2, BLOCKS_PER_TILE - 1):
        r = b % NRING
        wait(ssem[r], (rows_v[r], aggr_sh.at[idx_v[r].at[1]]))
    plsc.subcore_barrier()
    pltpu.sync_copy(aggr_sh.at[pl.ds(s * FR, FR)],
                    out_hbm.at[c, pl.ds(s * FR, FR)])


@functools.cache
def _edge_call():
    return pl.kernel(
        _edge_body,
        out_type=jax.ShapeDtypeStruct((NC, NP, H), jnp.float32),
        mesh=plsc.VectorSubcoreMesh(core_axis_name="c", subcore_axis_name="s",
                                    num_cores=NC, num_subcores=NS),
        compiler_params=pltpu.CompilerParams(needs_layout_passes=False),
        scratch_types=(
            [pltpu.VMEM((2, EB), jnp.int32) for _ in range(NRING)]
            + [pltpu.VMEM((EB * ED // 128, 128), jnp.float32)
               for _ in range(NRING)]
            + [pltpu.VMEM((EB, H), jnp.float32)
               for _ in range(NRING)]
            + [pltpu.VMEM((ED, H // 2), jnp.float32),
               pltpu.VMEM((H // 2,), jnp.float32)]
            + [pltpu.SemaphoreType.DMA for _ in range(4 * NRING)]
            + [pltpu.VMEM_SHARED((NP, H), jnp.float32)]
        ),
    )


def _node_lin_body(x_ref, w_ref, b_ref, o_ref):
    o_ref[...] = jnp.dot(x_ref[...], w_ref[...],
                         preferred_element_type=jnp.float32) + b_ref[...]


def _node_linear(x, w, b):
    fin = x.shape[1]
    return pl.pallas_call(
        _node_lin_body,
        grid=(N // RB,),
        in_specs=[pl.BlockSpec((RB, fin), lambda i: (i, 0)),
                  pl.BlockSpec((fin, H), lambda i: (0, 0)),
                  pl.BlockSpec((1, H), lambda i: (0, 0))],
        out_specs=pl.BlockSpec((RB, H), lambda i: (i, 0)),
        out_shape=jax.ShapeDtypeStruct((N, H), jnp.float32),
    )(x, w, b.reshape(1, H))


def _mlp_next_body(p_ref, xn_ref, m1_ref, b1_ref, m2_ref, b2_ref,
                   nw_ref, nb_ref, o_ref):
    aggr = p_ref[0] + p_ref[1] + jnp.maximum(xn_ref[...], 0.0)
    t = jnp.maximum(
        jnp.dot(aggr, m1_ref[...], preferred_element_type=jnp.float32)
        + b1_ref[...], 0.0)
    x1 = jnp.maximum(
        jnp.dot(t, m2_ref[...], preferred_element_type=jnp.float32)
        + b2_ref[...], 0.0)
    o_ref[...] = jnp.dot(x1, nw_ref[...],
                         preferred_element_type=jnp.float32) + nb_ref[...]


def _mlp_next(p, xn, m1, b1, m2, b2, nw, nb):
    return pl.pallas_call(
        _mlp_next_body,
        grid=(N // RB,),
        in_specs=[pl.BlockSpec((NC, RB, H), lambda i: (0, i, 0)),
                  pl.BlockSpec((RB, H), lambda i: (i, 0)),
                  pl.BlockSpec((H, H), lambda i: (0, 0)),
                  pl.BlockSpec((1, H), lambda i: (0, 0)),
                  pl.BlockSpec((H, H), lambda i: (0, 0)),
                  pl.BlockSpec((1, H), lambda i: (0, 0)),
                  pl.BlockSpec((H, H), lambda i: (0, 0)),
                  pl.BlockSpec((1, H), lambda i: (0, 0))],
        out_specs=pl.BlockSpec((RB, H), lambda i: (i, 0)),
        out_shape=jax.ShapeDtypeStruct((N, H), jnp.float32),
    )(p, xn, m1, b1.reshape(1, H), m2, b2.reshape(1, H), nw, nb.reshape(1, H))


def _mlp_pool_body(p_ref, xn_ref, m1_ref, b1_ref, m2_ref, b2_ref,
                   batch_ref, lw_ref, lb_ref, o_ref, acc_ref):
    i = pl.program_id(0)
    aggr = p_ref[0] + p_ref[1] + jnp.maximum(xn_ref[...], 0.0)
    t = jnp.maximum(
        jnp.dot(aggr, m1_ref[...], preferred_element_type=jnp.float32)
        + b1_ref[...], 0.0)
    x2 = jnp.maximum(
        jnp.dot(t, m2_ref[...], preferred_element_type=jnp.float32)
        + b2_ref[...], 0.0)
    bb = batch_ref[0, 0, :]
    onehot = (lax.broadcasted_iota(jnp.int32, (G, RB), 0)
              == bb[None, :]).astype(jnp.float32)
    pooled = jnp.dot(onehot, x2, preferred_element_type=jnp.float32)

    @pl.when(i == 0)
    def _():
        acc_ref[...] = jnp.zeros_like(acc_ref)

    acc_ref[...] += pooled

    @pl.when(i == N // RB - 1)
    def _():
        o_ref[...] = jnp.dot(acc_ref[...], lw_ref[...],
                             preferred_element_type=jnp.float32) + lb_ref[...]


def _mlp_pool(p, xn, m1, b1, m2, b2, batch3, lw, lb):
    return pl.pallas_call(
        _mlp_pool_body,
        grid=(N // RB,),
        in_specs=[pl.BlockSpec((NC, RB, H), lambda i: (0, i, 0)),
                  pl.BlockSpec((RB, H), lambda i: (i, 0)),
                  pl.BlockSpec((H, H), lambda i: (0, 0)),
                  pl.BlockSpec((1, H), lambda i: (0, 0)),
                  pl.BlockSpec((H, H), lambda i: (0, 0)),
                  pl.BlockSpec((1, H), lambda i: (0, 0)),
                  pl.BlockSpec((1, 1, RB), lambda i: (i, 0, 0)),
                  pl.BlockSpec((H, 1), lambda i: (0, 0)),
                  pl.BlockSpec((1, 1), lambda i: (0, 0))],
        out_specs=pl.BlockSpec((G, 1), lambda i: (0, 0)),
        out_shape=jax.ShapeDtypeStruct((G, 1), jnp.float32),
        scratch_shapes=[pltpu.VMEM((G, H), jnp.float32)],
    )(p, xn, m1, b1.reshape(1, H), m2, b2.reshape(1, H), batch3, lw,
      lb.reshape(1, 1))


def _shuffle_w(w):
    w4 = w.reshape(ED, CB, 2, LN)
    wb = w4.transpose(0, 1, 3, 2).reshape(ED, H // 2, 2).astype(jnp.bfloat16)
    return lax.bitcast_convert_type(wb, jnp.float32)


def _shuffle_b(b):
    b4 = b.reshape(CB, 2, LN)
    bb = b4.transpose(0, 2, 1).reshape(H // 2, 2).astype(jnp.bfloat16)
    return lax.bitcast_convert_type(bb, jnp.float32)


def kernel(pos, z, edge_index, edge_attr, batch,
           e1_node_W, e1_node_b, e1_edge_W, e1_edge_b,
           e1_m1_W, e1_m1_b, e1_m2_W, e1_m2_b,
           e2_node_W, e2_node_b, e2_edge_W, e2_edge_b,
           e2_m1_W, e2_m1_b, e2_m2_W, e2_m2_b,
           lin_W, lin_b):
    x0 = jnp.concatenate(
        [pos, jax.nn.one_hot(z, T, dtype=jnp.float32)], axis=1)
    src = edge_index[0]
    dst = edge_index[1]
    pad = E_PAD - E
    srcp = jnp.concatenate([src, jnp.zeros((pad,), src.dtype)]).reshape(
        NW, BLOCKS_PER_TILE, 1, EB)
    dstp = jnp.concatenate([dst, jnp.full((pad,), N, dst.dtype)]).reshape(
        NW, BLOCKS_PER_TILE, 1, EB)
    idxp = jnp.concatenate([srcp, dstp], axis=2)
    attrp = jnp.concatenate(
        [edge_attr, jnp.zeros((pad, ED), edge_attr.dtype)]).reshape(
        NW, BLOCKS_PER_TILE, EB * ED // 128, 128)
    zeros = jnp.zeros((ZR, H), jnp.float32)
    batch3 = batch.reshape(N // RB, 1, RB)

    xn1 = _node_linear(x0, e1_node_W, e1_node_b)
    p1 = _edge_call()(xn1, idxp, attrp, _shuffle_w(e1_edge_W),
                      _shuffle_b(e1_edge_b), zeros)
    xn2 = _mlp_next(p1, xn1, e1_m1_W, e1_m1_b, e1_m2_W, e1_m2_b,
                    e2_node_W, e2_node_b)
    p2 = _edge_call()(xn2, idxp, attrp, _shuffle_w(e2_edge_W),
                      _shuffle_b(e2_edge_b), zeros)
    return _mlp_pool(p2, xn2, e2_m1_W, e2_m1_b, e2_m2_W, e2_m2_b,
                     batch3, lin_W, lin_b)
